# baseline (speedup 1.0000x reference)
import functools

import jax
import jax.numpy as jnp
from jax import lax
from jax.experimental import pallas as pl
from jax.experimental.pallas import tpu as pltpu
from jax.experimental.pallas import tpu_sc as plsc

NC = 2
NS = 16
NW = NC * NS


def _deg_partials(dst, n_nodes):
    e = dst.shape[0]
    epw = e // NW
    ck = 80
    mesh = plsc.VectorSubcoreMesh(core_axis_name="c", subcore_axis_name="s")

    nchunk = epw // ck
    nq = 8

    @functools.partial(
        pl.kernel,
        out_type=jax.ShapeDtypeStruct((NC, n_nodes), jnp.float32),
        mesh=mesh,
        scratch_types=[
            pltpu.VMEM((nchunk, ck), jnp.int32),
            pltpu.VMEM((ck,), jnp.float32),
            pltpu.VMEM_SHARED((n_nodes,), jnp.float32),
            pltpu.SemaphoreType.DMA,
        ],
        compiler_params=pltpu.CompilerParams(use_tc_tiling_on_sc=False),
    )
    def k(dst_h, zeros_h, ones_h, out_h, idx_v, ones_v, deg_sh, sem):
        cid = lax.axis_index("c")
        sid = lax.axis_index("s")
        wid = cid * NS + sid

        @pl.when(sid == 0)
        def _():
            pltpu.sync_copy(zeros_h, deg_sh)

        pltpu.sync_copy(dst_h.at[pl.ds(wid * nchunk, nchunk)], idx_v)
        pltpu.sync_copy(ones_h, ones_v)
        plsc.subcore_barrier()

        def body(j, carry):
            pltpu.async_copy(ones_v, deg_sh.at[idx_v.at[j]], sem, add=True)

            @pl.when(j >= nq)
            def _():
                pltpu.make_async_copy(
                    ones_v, deg_sh.at[idx_v.at[0]], sem).wait()

            return carry

        lax.fori_loop(0, nchunk, body, 0)
        for _ in range(nq):
            pltpu.make_async_copy(ones_v, deg_sh.at[idx_v.at[0]], sem).wait()
        plsc.subcore_barrier()

        @pl.when(sid == 0)
        def _():
            pltpu.sync_copy(deg_sh, out_h.at[cid])

    return k(dst.reshape(NW * nchunk, ck), jnp.zeros((n_nodes,), jnp.float32),
             jnp.ones((ck,), jnp.float32))


def _propagate_partials(y, src, dst, ck, nbuf):
    n, d = y.shape
    e = src.shape[0]
    nchunk = -(e // -(ck * NW))
    epad = nchunk * ck * NW - e
    if epad:
        np_ = n + 8
        pad = jnp.full((epad,), n, dtype=src.dtype)
        src = jnp.concatenate([src, pad])
        dst = jnp.concatenate([dst, pad])
        y = jnp.concatenate([y, jnp.zeros((8, d), y.dtype)])
    else:
        np_ = n
    rows_per_tile = -(np_ // -NS) + 7 & ~7
    mesh = plsc.VectorSubcoreMesh(core_axis_name="c", subcore_axis_name="s")

    src2 = src.reshape(NW * nchunk, ck)
    dst2 = dst.reshape(NW * nchunk, ck)

    @functools.partial(
        pl.kernel,
        out_type=jax.ShapeDtypeStruct((NC, np_, d), jnp.float32),
        mesh=mesh,
        scratch_types=[
            pltpu.VMEM((nchunk, ck), jnp.int32),
            pltpu.VMEM((nchunk, ck), jnp.int32),
            [pltpu.VMEM((ck, d), jnp.float32)] * nbuf,
            pltpu.VMEM_SHARED((np_, d), jnp.float32),
            [pltpu.SemaphoreType.DMA] * nbuf,
            [pltpu.SemaphoreType.DMA] * nbuf,
        ],
        compiler_params=pltpu.CompilerParams(use_tc_tiling_on_sc=False),
    )
    def k(y_h, src_h, dst_h, zeros_h, out_h, srci_v, dsti_v, rows,
          acc_sh, semg, sems):
        cid = lax.axis_index("c")
        sid = lax.axis_index("s")
        wid = cid * NS + sid

        pltpu.sync_copy(src_h.at[pl.ds(wid * nchunk, nchunk)], srci_v)
        pltpu.sync_copy(dst_h.at[pl.ds(wid * nchunk, nchunk)], dsti_v)
        r0 = jnp.minimum(sid * rows_per_tile, np_ - rows_per_tile)
        pltpu.sync_copy(zeros_h.at[pl.ds(r0, rows_per_tile)],
                        acc_sh.at[pl.ds(r0, rows_per_tile)])
        plsc.subcore_barrier()

        def step(j, b):
            pltpu.make_async_copy(y_h.at[srci_v.at[j]], rows[b],
                                  semg[b]).wait()
            bn = (b + nbuf - 1) % nbuf

            @pl.when(j + nbuf - 1 < nchunk)
            def _():
                @pl.when(j >= 1)
                def _():
                    pltpu.make_async_copy(
                        rows[bn], acc_sh.at[dsti_v.at[0]], sems[bn]).wait()
                pltpu.async_copy(y_h.at[srci_v.at[j + nbuf - 1]], rows[bn],
                                 semg[bn])

            pltpu.async_copy(rows[b], acc_sh.at[dsti_v.at[j]], sems[b],
                             add=True)

        for b in range(nbuf - 1):
            pltpu.async_copy(y_h.at[srci_v.at[b]], rows[b], semg[b])

        def body(j, carry):
            for b in range(nbuf):
                @pl.when((j % nbuf) == b)
                def _(b=b):
                    step(j, b)
            return carry

        lax.fori_loop(0, nchunk, body, 0)
        for b in range(nbuf):
            pltpu.make_async_copy(rows[b], acc_sh.at[dsti_v.at[0]],
                                  sems[b]).wait()

        plsc.subcore_barrier()
        pltpu.sync_copy(acc_sh.at[pl.ds(r0, rows_per_tile)],
                        out_h.at[cid, pl.ds(r0, rows_per_tile)])

    return k(y, src2, dst2, jnp.zeros((np_, d), jnp.float32))


_BR = 1000


def _tc_matmul(x, w):
    n, din = x.shape
    dout = w.shape[1]

    def body(x_ref, w_ref, o_ref):
        o_ref[...] = jnp.dot(x_ref[...], w_ref[...],
                             preferred_element_type=jnp.float32)

    return pl.pallas_call(
        body,
        grid=(n // _BR,),
        in_specs=[
            pl.BlockSpec((_BR, din), lambda i: (i, 0)),
            pl.BlockSpec((din, dout), lambda i: (0, 0)),
        ],
        out_specs=pl.BlockSpec((_BR, dout), lambda i: (i, 0)),
        out_shape=jax.ShapeDtypeStruct((n, dout), jnp.float32),
    )(x, w)


def _tc_scale_in(xw, deg_t):
    n, dout = xw.shape

    def body(x_ref, dg_ref, y_ref, dis_ref):
        deg = dg_ref[:, 0:1] + dg_ref[:, 1:2] + 1.0
        dis = lax.rsqrt(deg)
        y_ref[...] = x_ref[...] * dis
        dis_ref[...] = dis

    return pl.pallas_call(
        body,
        grid=(n // _BR,),
        in_specs=[
            pl.BlockSpec((_BR, dout), lambda i: (i, 0)),
            pl.BlockSpec((_BR, 2), lambda i: (i, 0)),
        ],
        out_specs=[
            pl.BlockSpec((_BR, dout), lambda i: (i, 0)),
            pl.BlockSpec((_BR, 1), lambda i: (i, 0)),
        ],
        out_shape=[
            jax.ShapeDtypeStruct((n, dout), jnp.float32),
            jax.ShapeDtypeStruct((n, 1), jnp.float32),
        ],
    )(xw, deg_t)


def _tc_mid(p, y1, dis, b1, w2):
    n, dh = y1.shape
    dout = w2.shape[1]

    def body(p_ref, y_ref, dis_ref, b_ref, w_ref, o_ref):
        s = p_ref[0] + p_ref[1] + y_ref[...]
        h = jnp.maximum(s * dis_ref[...] + b_ref[...], 0.0)
        o_ref[...] = jnp.dot(h, w_ref[...],
                             preferred_element_type=jnp.float32) * dis_ref[...]

    return pl.pallas_call(
        body,
        grid=(n // _BR,),
        in_specs=[
            pl.BlockSpec((NC, _BR, dh), lambda i: (0, i, 0)),
            pl.BlockSpec((_BR, dh), lambda i: (i, 0)),
            pl.BlockSpec((_BR, 1), lambda i: (i, 0)),
            pl.BlockSpec((1, dh), lambda i: (0, 0)),
            pl.BlockSpec((dh, dout), lambda i: (0, 0)),
        ],
        out_specs=pl.BlockSpec((_BR, dout), lambda i: (i, 0)),
        out_shape=jax.ShapeDtypeStruct((n, dout), jnp.float32),
    )(p, y1, dis, b1.reshape(1, dh), w2)


def _tc_out(p, y2, dis, b2):
    n, dout = y2.shape

    def body(p_ref, y_ref, dis_ref, b_ref, o_ref):
        o = (p_ref[0] + p_ref[1] + y_ref[...]) * dis_ref[...] + b_ref[...]
        m = jnp.max(o, axis=-1, keepdims=True)
        ex = jnp.exp(o - m)
        o_ref[...] = ex / jnp.sum(ex, axis=-1, keepdims=True)

    return pl.pallas_call(
        body,
        grid=(n // _BR,),
        in_specs=[
            pl.BlockSpec((NC, _BR, dout), lambda i: (0, i, 0)),
            pl.BlockSpec((_BR, dout), lambda i: (i, 0)),
            pl.BlockSpec((_BR, 1), lambda i: (i, 0)),
            pl.BlockSpec((1, dout), lambda i: (0, 0)),
        ],
        out_specs=pl.BlockSpec((_BR, dout), lambda i: (i, 0)),
        out_shape=jax.ShapeDtypeStruct((n, dout), jnp.float32),
    )(p, y2, dis, b2.reshape(1, dout))


def kernel(x, edge_index, W1, b1, W2, b2):
    n = x.shape[0]
    src = edge_index[0]
    dst = edge_index[1]

    degp = _deg_partials(dst, n)
    xw1 = _tc_matmul(x, W1)
    deg_t = jnp.transpose(degp)

    y1, dis = _tc_scale_in(xw1, deg_t)
    p1 = _propagate_partials(y1, src, dst, ck=80, nbuf=3)
    y2 = _tc_mid(p1, y1, dis, b1, W2)
    p2 = _propagate_partials(y2, src, dst, ck=128, nbuf=6)
    return _tc_out(p2, y2, dis, b2)


# speedup vs baseline: 1.0011x; 1.0011x over previous
import functools

import jax
import jax.numpy as jnp
from jax import lax
from jax.experimental import pallas as pl
from jax.experimental.pallas import tpu as pltpu
from jax.experimental.pallas import tpu_sc as plsc

NC = 2
NS = 16
NW = NC * NS


def _deg_partials(dst, n_nodes):
    e = dst.shape[0]
    epw = e // NW
    ck = 80
    mesh = plsc.VectorSubcoreMesh(core_axis_name="c", subcore_axis_name="s")

    nchunk = epw // ck
    nq = 8

    @functools.partial(
        pl.kernel,
        out_type=jax.ShapeDtypeStruct((NC, n_nodes), jnp.float32),
        mesh=mesh,
        scratch_types=[
            pltpu.VMEM((nchunk, ck), jnp.int32),
            pltpu.VMEM((ck,), jnp.float32),
            pltpu.VMEM_SHARED((n_nodes,), jnp.float32),
            pltpu.SemaphoreType.DMA,
        ],
        compiler_params=pltpu.CompilerParams(use_tc_tiling_on_sc=False),
    )
    def k(dst_h, zeros_h, ones_h, out_h, idx_v, ones_v, deg_sh, sem):
        cid = lax.axis_index("c")
        sid = lax.axis_index("s")
        wid = cid * NS + sid

        @pl.when(sid == 0)
        def _():
            pltpu.sync_copy(zeros_h, deg_sh)

        pltpu.sync_copy(dst_h.at[pl.ds(wid * nchunk, nchunk)], idx_v)
        pltpu.sync_copy(ones_h, ones_v)
        plsc.subcore_barrier()

        def body(j, carry):
            pltpu.async_copy(ones_v, deg_sh.at[idx_v.at[j]], sem, add=True)

            @pl.when(j >= nq)
            def _():
                pltpu.make_async_copy(
                    ones_v, deg_sh.at[idx_v.at[0]], sem).wait()

            return carry

        lax.fori_loop(0, nchunk, body, 0)
        for _ in range(nq):
            pltpu.make_async_copy(ones_v, deg_sh.at[idx_v.at[0]], sem).wait()
        plsc.subcore_barrier()

        @pl.when(sid == 0)
        def _():
            pltpu.sync_copy(deg_sh, out_h.at[cid])

    return k(dst.reshape(NW * nchunk, ck), jnp.zeros((n_nodes,), jnp.float32),
             jnp.ones((ck,), jnp.float32))


def _propagate_partials(y, src, dst, ck, nbuf):
    n, d = y.shape
    e = src.shape[0]
    nchunk = -(e // -(ck * NW))
    epad = nchunk * ck * NW - e
    if epad:
        np_ = n + 128
        src = jnp.concatenate([src, jnp.full((epad,), n, dtype=src.dtype)])
        dst = jnp.concatenate(
            [dst, n + (jnp.arange(epad, dtype=dst.dtype) % 128)])
        y = jnp.concatenate([y, jnp.zeros((8, d), y.dtype)])
    else:
        np_ = n
    rows_per_tile = -(np_ // -NS) + 7 & ~7
    mesh = plsc.VectorSubcoreMesh(core_axis_name="c", subcore_axis_name="s")

    src2 = src.reshape(NW * nchunk, ck)
    dst2 = dst.reshape(NW * nchunk, ck)

    @functools.partial(
        pl.kernel,
        out_type=jax.ShapeDtypeStruct((NC, np_, d), jnp.float32),
        mesh=mesh,
        scratch_types=[
            pltpu.VMEM((nchunk, ck), jnp.int32),
            pltpu.VMEM((nchunk, ck), jnp.int32),
            [pltpu.VMEM((ck, d), jnp.float32)] * nbuf,
            pltpu.VMEM_SHARED((np_, d), jnp.float32),
            [pltpu.SemaphoreType.DMA] * nbuf,
            [pltpu.SemaphoreType.DMA] * nbuf,
        ],
        compiler_params=pltpu.CompilerParams(use_tc_tiling_on_sc=False),
    )
    def k(y_h, src_h, dst_h, zeros_h, out_h, srci_v, dsti_v, rows,
          acc_sh, semg, sems):
        cid = lax.axis_index("c")
        sid = lax.axis_index("s")
        wid = cid * NS + sid

        pltpu.sync_copy(src_h.at[pl.ds(wid * nchunk, nchunk)], srci_v)
        pltpu.sync_copy(dst_h.at[pl.ds(wid * nchunk, nchunk)], dsti_v)
        r0 = jnp.minimum(sid * rows_per_tile, np_ - rows_per_tile)
        pltpu.sync_copy(zeros_h.at[pl.ds(r0, rows_per_tile)],
                        acc_sh.at[pl.ds(r0, rows_per_tile)])
        plsc.subcore_barrier()

        def step(j, b):
            pltpu.make_async_copy(y_h.at[srci_v.at[j]], rows[b],
                                  semg[b]).wait()
            bn = (b + nbuf - 1) % nbuf

            @pl.when(j + nbuf - 1 < nchunk)
            def _():
                @pl.when(j >= 1)
                def _():
                    pltpu.make_async_copy(
                        rows[bn], acc_sh.at[dsti_v.at[0]], sems[bn]).wait()
                pltpu.async_copy(y_h.at[srci_v.at[j + nbuf - 1]], rows[bn],
                                 semg[bn])

            pltpu.async_copy(rows[b], acc_sh.at[dsti_v.at[j]], sems[b],
                             add=True)

        for b in range(nbuf - 1):
            pltpu.async_copy(y_h.at[srci_v.at[b]], rows[b], semg[b])

        def body(j, carry):
            for b in range(nbuf):
                @pl.when((j % nbuf) == b)
                def _(b=b):
                    step(j, b)
            return carry

        lax.fori_loop(0, nchunk, body, 0)
        for b in range(nbuf):
            pltpu.make_async_copy(rows[b], acc_sh.at[dsti_v.at[0]],
                                  sems[b]).wait()

        plsc.subcore_barrier()
        pltpu.sync_copy(acc_sh.at[pl.ds(r0, rows_per_tile)],
                        out_h.at[cid, pl.ds(r0, rows_per_tile)])

    return k(y, src2, dst2, jnp.zeros((np_, d), jnp.float32))


_BR = 1000


def _tc_matmul(x, w):
    n, din = x.shape
    dout = w.shape[1]

    def body(x_ref, w_ref, o_ref):
        o_ref[...] = jnp.dot(x_ref[...], w_ref[...],
                             preferred_element_type=jnp.float32)

    return pl.pallas_call(
        body,
        grid=(n // _BR,),
        in_specs=[
            pl.BlockSpec((_BR, din), lambda i: (i, 0)),
            pl.BlockSpec((din, dout), lambda i: (0, 0)),
        ],
        out_specs=pl.BlockSpec((_BR, dout), lambda i: (i, 0)),
        out_shape=jax.ShapeDtypeStruct((n, dout), jnp.float32),
    )(x, w)


def _tc_scale_in(xw, deg_t):
    n, dout = xw.shape

    def body(x_ref, dg_ref, y_ref, dis_ref):
        deg = dg_ref[:, 0:1] + dg_ref[:, 1:2] + 1.0
        dis = lax.rsqrt(deg)
        y_ref[...] = x_ref[...] * dis
        dis_ref[...] = dis

    return pl.pallas_call(
        body,
        grid=(n // _BR,),
        in_specs=[
            pl.BlockSpec((_BR, dout), lambda i: (i, 0)),
            pl.BlockSpec((_BR, 2), lambda i: (i, 0)),
        ],
        out_specs=[
            pl.BlockSpec((_BR, dout), lambda i: (i, 0)),
            pl.BlockSpec((_BR, 1), lambda i: (i, 0)),
        ],
        out_shape=[
            jax.ShapeDtypeStruct((n, dout), jnp.float32),
            jax.ShapeDtypeStruct((n, 1), jnp.float32),
        ],
    )(xw, deg_t)


def _tc_mid(p, y1, dis, b1, w2):
    n, dh = y1.shape
    dout = w2.shape[1]

    def body(p_ref, y_ref, dis_ref, b_ref, w_ref, o_ref):
        s = p_ref[0] + p_ref[1] + y_ref[...]
        h = jnp.maximum(s * dis_ref[...] + b_ref[...], 0.0)
        o_ref[...] = jnp.dot(h, w_ref[...],
                             preferred_element_type=jnp.float32) * dis_ref[...]

    return pl.pallas_call(
        body,
        grid=(n // _BR,),
        in_specs=[
            pl.BlockSpec((NC, _BR, dh), lambda i: (0, i, 0)),
            pl.BlockSpec((_BR, dh), lambda i: (i, 0)),
            pl.BlockSpec((_BR, 1), lambda i: (i, 0)),
            pl.BlockSpec((1, dh), lambda i: (0, 0)),
            pl.BlockSpec((dh, dout), lambda i: (0, 0)),
        ],
        out_specs=pl.BlockSpec((_BR, dout), lambda i: (i, 0)),
        out_shape=jax.ShapeDtypeStruct((n, dout), jnp.float32),
    )(p, y1, dis, b1.reshape(1, dh), w2)


def _tc_out(p, y2, dis, b2):
    n, dout = y2.shape

    def body(p_ref, y_ref, dis_ref, b_ref, o_ref):
        o = (p_ref[0] + p_ref[1] + y_ref[...]) * dis_ref[...] + b_ref[...]
        m = jnp.max(o, axis=-1, keepdims=True)
        ex = jnp.exp(o - m)
        o_ref[...] = ex / jnp.sum(ex, axis=-1, keepdims=True)

    return pl.pallas_call(
        body,
        grid=(n // _BR,),
        in_specs=[
            pl.BlockSpec((NC, _BR, dout), lambda i: (0, i, 0)),
            pl.BlockSpec((_BR, dout), lambda i: (i, 0)),
            pl.BlockSpec((_BR, 1), lambda i: (i, 0)),
            pl.BlockSpec((1, dout), lambda i: (0, 0)),
        ],
        out_specs=pl.BlockSpec((_BR, dout), lambda i: (i, 0)),
        out_shape=jax.ShapeDtypeStruct((n, dout), jnp.float32),
    )(p, y2, dis, b2.reshape(1, dout))


def kernel(x, edge_index, W1, b1, W2, b2):
    n = x.shape[0]
    src = edge_index[0]
    dst = edge_index[1]

    degp = _deg_partials(dst, n)
    xw1 = _tc_matmul(x, W1)
    deg_t = jnp.transpose(degp)

    y1, dis = _tc_scale_in(xw1, deg_t)
    p1 = _propagate_partials(y1, src, dst, ck=80, nbuf=3)
    y2 = _tc_mid(p1, y1, dis, b1, W2)
    p2 = _propagate_partials(y2, src, dst, ck=128, nbuf=6)
    return _tc_out(p2, y2, dis, b2)


# speedup vs baseline: 1.2204x; 1.2190x over previous
import functools

import jax
import jax.numpy as jnp
from jax import lax
from jax.experimental import pallas as pl
from jax.experimental.pallas import tpu as pltpu
from jax.experimental.pallas import tpu_sc as plsc

NC = 2
NS = 16
NW = NC * NS


def _deg_partials(dst, n_nodes):
    e = dst.shape[0]
    epw = e // NW
    ck = 80
    mesh = plsc.VectorSubcoreMesh(core_axis_name="c", subcore_axis_name="s")

    nchunk = epw // ck
    nq = 8

    @functools.partial(
        pl.kernel,
        out_type=jax.ShapeDtypeStruct((NC, n_nodes), jnp.float32),
        mesh=mesh,
        scratch_types=[
            pltpu.VMEM((nchunk, ck), jnp.int32),
            pltpu.VMEM((ck,), jnp.float32),
            pltpu.VMEM_SHARED((n_nodes,), jnp.float32),
            pltpu.SemaphoreType.DMA,
        ],
        compiler_params=pltpu.CompilerParams(use_tc_tiling_on_sc=False),
    )
    def k(dst_h, zeros_h, ones_h, out_h, idx_v, ones_v, deg_sh, sem):
        cid = lax.axis_index("c")
        sid = lax.axis_index("s")
        wid = cid * NS + sid

        @pl.when(sid == 0)
        def _():
            pltpu.sync_copy(zeros_h, deg_sh)

        pltpu.sync_copy(dst_h.at[pl.ds(wid * nchunk, nchunk)], idx_v)
        pltpu.sync_copy(ones_h, ones_v)
        plsc.subcore_barrier()

        def body(j, carry):
            pltpu.async_copy(ones_v, deg_sh.at[idx_v.at[j]], sem, add=True)

            @pl.when(j >= nq)
            def _():
                pltpu.make_async_copy(
                    ones_v, deg_sh.at[idx_v.at[0]], sem).wait()

            return carry

        lax.fori_loop(0, nchunk, body, 0)
        for _ in range(nq):
            pltpu.make_async_copy(ones_v, deg_sh.at[idx_v.at[0]], sem).wait()
        plsc.subcore_barrier()

        @pl.when(sid == 0)
        def _():
            pltpu.sync_copy(deg_sh, out_h.at[cid])

    return k(dst.reshape(NW * nchunk, ck), jnp.zeros((n_nodes,), jnp.float32),
             jnp.ones((ck,), jnp.float32))


def _propagate_partials(y, src, dst, ck, nbuf):
    n, d = y.shape
    e = src.shape[0]
    nchunk = -(e // -(ck * NW))
    epad = nchunk * ck * NW - e
    if epad:
        np_ = n + 128
        idx_pad = jnp.arange(epad, dtype=src.dtype)
        src = jnp.concatenate([src, idx_pad % n])
        dst = jnp.concatenate([dst, n + (idx_pad % 128)])
    else:
        np_ = n
    rows_per_tile = -(np_ // -NS) + 7 & ~7
    mesh = plsc.VectorSubcoreMesh(core_axis_name="c", subcore_axis_name="s")

    src2 = src.reshape(NW * nchunk, ck)
    dst2 = dst.reshape(NW * nchunk, ck)

    @functools.partial(
        pl.kernel,
        out_type=jax.ShapeDtypeStruct((NC, np_, d), jnp.float32),
        mesh=mesh,
        scratch_types=[
            pltpu.VMEM((nchunk, ck), jnp.int32),
            pltpu.VMEM((nchunk, ck), jnp.int32),
            [pltpu.VMEM((ck, d), jnp.float32)] * nbuf,
            pltpu.VMEM_SHARED((np_, d), jnp.float32),
            [pltpu.SemaphoreType.DMA] * nbuf,
            [pltpu.SemaphoreType.DMA] * nbuf,
        ],
        compiler_params=pltpu.CompilerParams(use_tc_tiling_on_sc=False),
    )
    def k(y_h, src_h, dst_h, zeros_h, out_h, srci_v, dsti_v, rows,
          acc_sh, semg, sems):
        cid = lax.axis_index("c")
        sid = lax.axis_index("s")
        wid = cid * NS + sid

        pltpu.sync_copy(src_h.at[pl.ds(wid * nchunk, nchunk)], srci_v)
        pltpu.sync_copy(dst_h.at[pl.ds(wid * nchunk, nchunk)], dsti_v)
        r0 = jnp.minimum(sid * rows_per_tile, np_ - rows_per_tile)
        pltpu.sync_copy(zeros_h.at[pl.ds(r0, rows_per_tile)],
                        acc_sh.at[pl.ds(r0, rows_per_tile)])
        plsc.subcore_barrier()

        def step(j, b):
            pltpu.make_async_copy(y_h.at[srci_v.at[j]], rows[b],
                                  semg[b]).wait()
            bn = (b + nbuf - 1) % nbuf

            @pl.when(j + nbuf - 1 < nchunk)
            def _():
                @pl.when(j >= 1)
                def _():
                    pltpu.make_async_copy(
                        rows[bn], acc_sh.at[dsti_v.at[0]], sems[bn]).wait()
                pltpu.async_copy(y_h.at[srci_v.at[j + nbuf - 1]], rows[bn],
                                 semg[bn])

            pltpu.async_copy(rows[b], acc_sh.at[dsti_v.at[j]], sems[b],
                             add=True)

        for b in range(nbuf - 1):
            pltpu.async_copy(y_h.at[srci_v.at[b]], rows[b], semg[b])

        def body(j, carry):
            for b in range(nbuf):
                @pl.when((j % nbuf) == b)
                def _(b=b):
                    step(j, b)
            return carry

        lax.fori_loop(0, nchunk, body, 0)
        for b in range(nbuf):
            pltpu.make_async_copy(rows[b], acc_sh.at[dsti_v.at[0]],
                                  sems[b]).wait()

        plsc.subcore_barrier()
        pltpu.sync_copy(acc_sh.at[pl.ds(r0, rows_per_tile)],
                        out_h.at[cid, pl.ds(r0, rows_per_tile)])

    return k(y, src2, dst2, jnp.zeros((np_, d), jnp.float32))


_BR = 1000


def _tc_matmul(x, w):
    n, din = x.shape
    dout = w.shape[1]

    def body(x_ref, w_ref, o_ref):
        o_ref[...] = jnp.dot(x_ref[...], w_ref[...],
                             preferred_element_type=jnp.float32)

    return pl.pallas_call(
        body,
        grid=(n // _BR,),
        in_specs=[
            pl.BlockSpec((_BR, din), lambda i: (i, 0)),
            pl.BlockSpec((din, dout), lambda i: (0, 0)),
        ],
        out_specs=pl.BlockSpec((_BR, dout), lambda i: (i, 0)),
        out_shape=jax.ShapeDtypeStruct((n, dout), jnp.float32),
    )(x, w)


def _tc_scale_in(xw, deg_t):
    n, dout = xw.shape

    def body(x_ref, dg_ref, y_ref, dis_ref):
        deg = dg_ref[:, 0:1] + dg_ref[:, 1:2] + 1.0
        dis = lax.rsqrt(deg)
        y_ref[...] = x_ref[...] * dis
        dis_ref[...] = dis

    return pl.pallas_call(
        body,
        grid=(n // _BR,),
        in_specs=[
            pl.BlockSpec((_BR, dout), lambda i: (i, 0)),
            pl.BlockSpec((_BR, 2), lambda i: (i, 0)),
        ],
        out_specs=[
            pl.BlockSpec((_BR, dout), lambda i: (i, 0)),
            pl.BlockSpec((_BR, 1), lambda i: (i, 0)),
        ],
        out_shape=[
            jax.ShapeDtypeStruct((n, dout), jnp.float32),
            jax.ShapeDtypeStruct((n, 1), jnp.float32),
        ],
    )(xw, deg_t)


def _tc_mid(p, y1, dis, b1, w2):
    n, dh = y1.shape
    dout = w2.shape[1]

    def body(p_ref, y_ref, dis_ref, b_ref, w_ref, o_ref):
        s = p_ref[0] + p_ref[1] + y_ref[...]
        h = jnp.maximum(s * dis_ref[...] + b_ref[...], 0.0)
        o_ref[...] = jnp.dot(h, w_ref[...],
                             preferred_element_type=jnp.float32) * dis_ref[...]

    return pl.pallas_call(
        body,
        grid=(n // _BR,),
        in_specs=[
            pl.BlockSpec((NC, _BR, dh), lambda i: (0, i, 0)),
            pl.BlockSpec((_BR, dh), lambda i: (i, 0)),
            pl.BlockSpec((_BR, 1), lambda i: (i, 0)),
            pl.BlockSpec((1, dh), lambda i: (0, 0)),
            pl.BlockSpec((dh, dout), lambda i: (0, 0)),
        ],
        out_specs=pl.BlockSpec((_BR, dout), lambda i: (i, 0)),
        out_shape=jax.ShapeDtypeStruct((n, dout), jnp.float32),
    )(p, y1, dis, b1.reshape(1, dh), w2)


def _tc_out(p, y2, dis, b2):
    n, dout = y2.shape

    def body(p_ref, y_ref, dis_ref, b_ref, o_ref):
        o = (p_ref[0] + p_ref[1] + y_ref[...]) * dis_ref[...] + b_ref[...]
        m = jnp.max(o, axis=-1, keepdims=True)
        ex = jnp.exp(o - m)
        o_ref[...] = ex / jnp.sum(ex, axis=-1, keepdims=True)

    return pl.pallas_call(
        body,
        grid=(n // _BR,),
        in_specs=[
            pl.BlockSpec((NC, _BR, dout), lambda i: (0, i, 0)),
            pl.BlockSpec((_BR, dout), lambda i: (i, 0)),
            pl.BlockSpec((_BR, 1), lambda i: (i, 0)),
            pl.BlockSpec((1, dout), lambda i: (0, 0)),
        ],
        out_specs=pl.BlockSpec((_BR, dout), lambda i: (i, 0)),
        out_shape=jax.ShapeDtypeStruct((n, dout), jnp.float32),
    )(p, y2, dis, b2.reshape(1, dout))


def kernel(x, edge_index, W1, b1, W2, b2):
    n = x.shape[0]
    src = edge_index[0]
    dst = edge_index[1]

    degp = _deg_partials(dst, n)
    xw1 = _tc_matmul(x, W1)
    deg_t = jnp.transpose(degp)

    y1, dis = _tc_scale_in(xw1, deg_t)
    p1 = _propagate_partials(y1, src, dst, ck=80, nbuf=3)
    y2 = _tc_mid(p1, y1, dis, b1, W2)
    p2 = _propagate_partials(y2, src, dst, ck=128, nbuf=6)
    return _tc_out(p2, y2, dis, b2)


# speedup vs baseline: 1.3147x; 1.0773x over previous
import functools

import jax
import jax.numpy as jnp
from jax import lax
from jax.experimental import pallas as pl
from jax.experimental.pallas import tpu as pltpu
from jax.experimental.pallas import tpu_sc as plsc

NC = 2
NS = 16
NW = NC * NS


def _deg_partials(ei, n_nodes):
    nchunk = ei.shape[1] // NW
    ck = ei.shape[2]
    mesh = plsc.VectorSubcoreMesh(core_axis_name="c", subcore_axis_name="s")

    nq = 8

    @functools.partial(
        pl.kernel,
        out_type=jax.ShapeDtypeStruct((NC, n_nodes), jnp.float32),
        mesh=mesh,
        scratch_types=[
            pltpu.VMEM((nchunk, ck), jnp.int32),
            pltpu.VMEM((ck,), jnp.float32),
            pltpu.VMEM_SHARED((n_nodes,), jnp.float32),
            pltpu.SemaphoreType.DMA,
        ],
        compiler_params=pltpu.CompilerParams(use_tc_tiling_on_sc=False),
    )
    def k(ei_h, zeros_h, ones_h, out_h, idx_v, ones_v, deg_sh, sem):
        cid = lax.axis_index("c")
        sid = lax.axis_index("s")
        wid = cid * NS + sid

        @pl.when(sid == 0)
        def _():
            pltpu.sync_copy(zeros_h, deg_sh)

        pltpu.sync_copy(ei_h.at[1, pl.ds(wid * nchunk, nchunk)], idx_v)
        pltpu.sync_copy(ones_h, ones_v)
        plsc.subcore_barrier()

        def body(j, carry):
            pltpu.async_copy(ones_v, deg_sh.at[idx_v.at[j]], sem, add=True)

            @pl.when(j >= nq)
            def _():
                pltpu.make_async_copy(
                    ones_v, deg_sh.at[idx_v.at[0]], sem).wait()

            return carry

        lax.fori_loop(0, nchunk, body, 0)
        for _ in range(nq):
            pltpu.make_async_copy(ones_v, deg_sh.at[idx_v.at[0]], sem).wait()
        plsc.subcore_barrier()

        @pl.when(sid == 0)
        def _():
            pltpu.sync_copy(deg_sh, out_h.at[cid])

    return k(ei, jnp.zeros((n_nodes,), jnp.float32),
             jnp.ones((ck,), jnp.float32))


def _propagate_partials(y, ei, np_, nbuf):
    n, d = y.shape
    nchunk = ei.shape[1] // NW
    ck = ei.shape[2]
    rows_per_tile = -(np_ // -NS) + 7 & ~7
    mesh = plsc.VectorSubcoreMesh(core_axis_name="c", subcore_axis_name="s")

    @functools.partial(
        pl.kernel,
        out_type=jax.ShapeDtypeStruct((NC, np_, d), jnp.float32),
        mesh=mesh,
        scratch_types=[
            pltpu.VMEM((nchunk, ck), jnp.int32),
            pltpu.VMEM((nchunk, ck), jnp.int32),
            [pltpu.VMEM((ck, d), jnp.float32)] * nbuf,
            pltpu.VMEM_SHARED((np_, d), jnp.float32),
            [pltpu.SemaphoreType.DMA] * nbuf,
            [pltpu.SemaphoreType.DMA] * nbuf,
        ],
        compiler_params=pltpu.CompilerParams(use_tc_tiling_on_sc=False),
    )
    def k(y_h, ei_h, zeros_h, out_h, srci_v, dsti_v, rows,
          acc_sh, semg, sems):
        cid = lax.axis_index("c")
        sid = lax.axis_index("s")
        wid = cid * NS + sid

        pltpu.sync_copy(ei_h.at[0, pl.ds(wid * nchunk, nchunk)], srci_v)
        pltpu.sync_copy(ei_h.at[1, pl.ds(wid * nchunk, nchunk)], dsti_v)
        r0 = jnp.minimum(sid * rows_per_tile, np_ - rows_per_tile)
        pltpu.sync_copy(zeros_h, acc_sh.at[pl.ds(r0, rows_per_tile)])
        plsc.subcore_barrier()

        def step(j, b):
            pltpu.make_async_copy(y_h.at[srci_v.at[j]], rows[b],
                                  semg[b]).wait()
            bn = (b + nbuf - 1) % nbuf

            @pl.when(j + nbuf - 1 < nchunk)
            def _():
                @pl.when(j >= 1)
                def _():
                    pltpu.make_async_copy(
                        rows[bn], acc_sh.at[dsti_v.at[0]], sems[bn]).wait()
                pltpu.async_copy(y_h.at[srci_v.at[j + nbuf - 1]], rows[bn],
                                 semg[bn])

            pltpu.async_copy(rows[b], acc_sh.at[dsti_v.at[j]], sems[b],
                             add=True)

        for b in range(nbuf - 1):
            pltpu.async_copy(y_h.at[srci_v.at[b]], rows[b], semg[b])

        def body(j, carry):
            for b in range(nbuf):
                @pl.when((j % nbuf) == b)
                def _(b=b):
                    step(j, b)
            return carry

        lax.fori_loop(0, nchunk, body, 0)
        for b in range(nbuf):
            pltpu.make_async_copy(rows[b], acc_sh.at[dsti_v.at[0]],
                                  sems[b]).wait()

        plsc.subcore_barrier()
        pltpu.sync_copy(acc_sh.at[pl.ds(r0, rows_per_tile)],
                        out_h.at[cid, pl.ds(r0, rows_per_tile)])

    return k(y, ei, jnp.zeros((rows_per_tile, d), jnp.float32))


_BR = 2000


def _tc_matmul(x, w):
    n, din = x.shape
    dout = w.shape[1]

    def body(x_ref, w_ref, o_ref):
        o_ref[...] = jnp.dot(x_ref[...], w_ref[...],
                             preferred_element_type=jnp.float32)

    return pl.pallas_call(
        body,
        grid=(n // _BR,),
        in_specs=[
            pl.BlockSpec((_BR, din), lambda i: (i, 0)),
            pl.BlockSpec((din, dout), lambda i: (0, 0)),
        ],
        out_specs=pl.BlockSpec((_BR, dout), lambda i: (i, 0)),
        out_shape=jax.ShapeDtypeStruct((n, dout), jnp.float32),
    )(x, w)


def _tc_scale_in(xw, deg_t):
    n, dout = xw.shape

    def body(x_ref, dg_ref, y_ref, dis_ref):
        deg = dg_ref[:, 0:1] + dg_ref[:, 1:2] + 1.0
        dis = lax.rsqrt(deg)
        y_ref[...] = x_ref[...] * dis
        dis_ref[...] = dis

    return pl.pallas_call(
        body,
        grid=(n // _BR,),
        in_specs=[
            pl.BlockSpec((_BR, dout), lambda i: (i, 0)),
            pl.BlockSpec((_BR, 2), lambda i: (i, 0)),
        ],
        out_specs=[
            pl.BlockSpec((_BR, dout), lambda i: (i, 0)),
            pl.BlockSpec((_BR, 1), lambda i: (i, 0)),
        ],
        out_shape=[
            jax.ShapeDtypeStruct((n, dout), jnp.float32),
            jax.ShapeDtypeStruct((n, 1), jnp.float32),
        ],
    )(xw, deg_t)


def _tc_mid(p, y1, dis, b1, w2):
    n, dh = y1.shape
    dout = w2.shape[1]

    def body(p_ref, y_ref, dis_ref, b_ref, w_ref, o_ref):
        s = p_ref[0] + p_ref[1] + y_ref[...]
        h = jnp.maximum(s * dis_ref[...] + b_ref[...], 0.0)
        o_ref[...] = jnp.dot(h, w_ref[...],
                             preferred_element_type=jnp.float32) * dis_ref[...]

    return pl.pallas_call(
        body,
        grid=(n // _BR,),
        in_specs=[
            pl.BlockSpec((NC, _BR, dh), lambda i: (0, i, 0)),
            pl.BlockSpec((_BR, dh), lambda i: (i, 0)),
            pl.BlockSpec((_BR, 1), lambda i: (i, 0)),
            pl.BlockSpec((1, dh), lambda i: (0, 0)),
            pl.BlockSpec((dh, dout), lambda i: (0, 0)),
        ],
        out_specs=pl.BlockSpec((_BR, dout), lambda i: (i, 0)),
        out_shape=jax.ShapeDtypeStruct((n, dout), jnp.float32),
    )(p, y1, dis, b1.reshape(1, dh), w2)


def _tc_out(p, y2, dis, b2):
    n, dout = y2.shape

    def body(p_ref, y_ref, dis_ref, b_ref, o_ref):
        o = (p_ref[0] + p_ref[1] + y_ref[...]) * dis_ref[...] + b_ref[...]
        m = jnp.max(o, axis=-1, keepdims=True)
        ex = jnp.exp(o - m)
        o_ref[...] = ex / jnp.sum(ex, axis=-1, keepdims=True)

    return pl.pallas_call(
        body,
        grid=(n // _BR,),
        in_specs=[
            pl.BlockSpec((NC, _BR, dout), lambda i: (0, i, 0)),
            pl.BlockSpec((_BR, dout), lambda i: (i, 0)),
            pl.BlockSpec((_BR, 1), lambda i: (i, 0)),
            pl.BlockSpec((1, dout), lambda i: (0, 0)),
        ],
        out_specs=pl.BlockSpec((_BR, dout), lambda i: (i, 0)),
        out_shape=jax.ShapeDtypeStruct((n, dout), jnp.float32),
    )(p, y2, dis, b2.reshape(1, dout))


def kernel(x, edge_index, W1, b1, W2, b2):
    n = x.shape[0]
    e = edge_index.shape[1]

    ck1, ck2 = 80, 128
    ei1 = edge_index.reshape(2, e // ck1, ck1)
    nch2 = -(e // -(ck2 * NW))
    epad = nch2 * ck2 * NW - e
    idx_pad = jnp.arange(epad, dtype=edge_index.dtype)
    ei2 = jnp.concatenate(
        [edge_index, jnp.stack([idx_pad % n, n + (idx_pad % 128)])],
        axis=1).reshape(2, NW * nch2, ck2)

    degp = _deg_partials(ei1, n)
    xw1 = _tc_matmul(x, W1)
    deg_t = jnp.transpose(degp)

    y1, dis = _tc_scale_in(xw1, deg_t)
    p1 = _propagate_partials(y1, ei1, n, nbuf=3)
    y2 = _tc_mid(p1, y1, dis, b1, W2)
    p2 = _propagate_partials(y2, ei2, n + 128, nbuf=6)
    return _tc_out(p2, y2, dis, b2)
